# baseline (speedup 1.0000x reference)
import functools

import jax
import jax.numpy as jnp
from jax import lax
from jax.experimental import pallas as pl
from jax.experimental.pallas import tpu as pltpu
from jax.experimental.pallas import tpu_sc as plsc

_TOPK = 2
_T = 256
_FT = 1024
_RT = 512
_INTERPRET = False


def _router_body(x_ref, wg_ref, bg_ref, eidx_ref, wgt_ref, cnt_ref):
    step = pl.program_id(0)
    x = x_ref[...]
    logits = jnp.dot(x, wg_ref[...], preferred_element_type=jnp.float32)
    logits = logits + bg_ref[...]
    m = jnp.max(logits, axis=1, keepdims=True)
    ex = jnp.exp(logits - m)
    probs = ex / jnp.sum(ex, axis=1, keepdims=True)
    lane = jax.lax.broadcasted_iota(jnp.int32, probs.shape, 1)
    big = jnp.int32(10**6)
    m0 = jnp.max(probs, axis=1, keepdims=True)
    i0 = jnp.min(jnp.where(probs == m0, lane, big), axis=1, keepdims=True)
    probs1 = jnp.where(lane == i0, -1.0, probs)
    m1 = jnp.max(probs1, axis=1, keepdims=True)
    i1 = jnp.min(jnp.where(probs1 == m1, lane, big), axis=1, keepdims=True)
    eidx_ref[...] = jnp.where(lane == 0, i0, jnp.where(lane == 1, i1, 0))
    wgt_ref[...] = jnp.where(lane == 0, m0, jnp.where(lane == 1, m1, 0.0))
    oh = ((lane == i0) | (lane == i1)).astype(jnp.int32)
    chunk_counts = jnp.sum(oh, axis=0, keepdims=True)

    @pl.when(step == 0)
    def _():
        cnt_ref[...] = chunk_counts

    @pl.when(step > 0)
    def _():
        cnt_ref[...] = cnt_ref[...] + chunk_counts


def _router(x_flat, Wg, bg):
    n, d = x_flat.shape
    e = Wg.shape[1]
    wg_pad = jnp.zeros((d, 128), jnp.float32).at[:, :e].set(Wg)
    bg_pad = jnp.full((1, 128), -1e30, jnp.float32).at[0, :e].set(bg)
    return pl.pallas_call(
        _router_body,
        grid=(n // _RT,),
        in_specs=[
            pl.BlockSpec((_RT, d), lambda i: (i, 0)),
            pl.BlockSpec((d, 128), lambda i: (0, 0)),
            pl.BlockSpec((1, 128), lambda i: (0, 0)),
        ],
        out_specs=[
            pl.BlockSpec((_RT, 128), lambda i: (i, 0)),
            pl.BlockSpec((_RT, 128), lambda i: (i, 0)),
            pl.BlockSpec((1, 128), lambda i: (0, 0)),
        ],
        out_shape=[
            jax.ShapeDtypeStruct((n, 128), jnp.int32),
            jax.ShapeDtypeStruct((n, 128), jnp.float32),
            jax.ShapeDtypeStruct((1, 128), jnp.int32),
        ],
        compiler_params=pltpu.CompilerParams(
            dimension_semantics=("arbitrary",)),
        interpret=_INTERPRET,
    )(x_flat, wg_pad, bg_pad)


def _slot_body(eidx_ref, seg_ref, slot_ref, run_ref):
    step = pl.program_id(0)

    @pl.when(step == 0)
    def _():
        run_ref[...] = jnp.zeros_like(run_ref)

    eidx = eidx_ref[...]
    lane = jax.lax.broadcasted_iota(jnp.int32, eidx.shape, 1)
    i0 = jnp.sum(jnp.where(lane == 0, eidx, 0), axis=1, keepdims=True)
    i1 = jnp.sum(jnp.where(lane == 1, eidx, 0), axis=1, keepdims=True)
    oh0 = (lane == i0).astype(jnp.float32)
    oh1 = (lane == i1).astype(jnp.float32)
    oh = jnp.concatenate([oh0, oh1], axis=0)
    m = 2 * _RT
    r = jax.lax.broadcasted_iota(jnp.int32, (m, m), 0)
    c = jax.lax.broadcasted_iota(jnp.int32, (m, m), 1)
    ltri = (r > c).astype(jnp.float32)
    pref = jnp.dot(ltri, oh, preferred_element_type=jnp.float32)
    base = run_ref[...] + seg_ref[...]
    slot_pe = jnp.sum(oh * (pref + base), axis=1, keepdims=True)
    s0 = slot_pe[:_RT]
    s1 = slot_pe[_RT:]
    slot_ref[...] = jnp.where(
        lane == 0, s0, jnp.where(lane == 1, s1, 0.0)).astype(jnp.int32)
    run_ref[...] = run_ref[...] + jnp.sum(oh, axis=0, keepdims=True)


def _slot_assign(eidx, seg):
    n = eidx.shape[0]
    return pl.pallas_call(
        _slot_body,
        grid=(n // _RT,),
        in_specs=[
            pl.BlockSpec((_RT, 128), lambda i: (i, 0)),
            pl.BlockSpec((1, 128), lambda i: (0, 0)),
        ],
        out_specs=pl.BlockSpec((_RT, 128), lambda i: (i, 0)),
        out_shape=jax.ShapeDtypeStruct((n, 128), jnp.int32),
        scratch_shapes=[pltpu.VMEM((1, 128), jnp.float32)],
        compiler_params=pltpu.CompilerParams(
            dimension_semantics=("arbitrary",)),
        interpret=_INTERPRET,
    )(eidx, seg)


def _erf(z):
    s = jnp.sign(z)
    a = jnp.abs(z)
    t = 1.0 / (1.0 + 0.3275911 * a)
    poly = t * (0.254829592 + t * (-0.284496736 + t * (1.421413741
           + t * (-1.453152027 + t * 1.061405429))))
    return s * (1.0 - poly * jnp.exp(-a * a))


def _gelu(h):
    return 0.5 * h * (1.0 + _erf(h * 0.7071067811865476))


def _ffn_body(nf, be_ref, xs_ref, w1_ref, b1_ref, w2_ref, b2_ref, gw_ref,
              out_ref):
    f = pl.program_id(1)
    xb = xs_ref[...].astype(jnp.bfloat16)
    h = jnp.dot(xb, w1_ref[0], preferred_element_type=jnp.float32)
    h = _gelu(h + b1_ref[0])
    acc = jnp.dot(h.astype(jnp.bfloat16), w2_ref[0],
                  preferred_element_type=jnp.float32)

    @pl.when(f == 0)
    def _():
        out_ref[...] = acc

    @pl.when(f > 0)
    def _():
        out_ref[...] = out_ref[...] + acc

    @pl.when(f == nf - 1)
    def _():
        out_ref[...] = (out_ref[...] + b2_ref[0]) * gw_ref[...]


def _ffn(xs, W1, b1, W2, b2, gw, be):
    ns, d = xs.shape
    e, _, f_dim = W1.shape
    nb = ns // _T
    nf = f_dim // _FT
    grid_spec = pltpu.PrefetchScalarGridSpec(
        num_scalar_prefetch=1,
        grid=(nb, nf),
        in_specs=[
            pl.BlockSpec((_T, d), lambda b, f, be: (b, 0)),
            pl.BlockSpec((1, d, _FT), lambda b, f, be: (be[b], 0, f)),
            pl.BlockSpec((1, 1, _FT), lambda b, f, be: (be[b], 0, f)),
            pl.BlockSpec((1, _FT, d), lambda b, f, be: (be[b], f, 0)),
            pl.BlockSpec((1, 1, d), lambda b, f, be: (be[b], 0, 0)),
            pl.BlockSpec((_T, 1), lambda b, f, be: (b, 0)),
        ],
        out_specs=pl.BlockSpec((_T, d), lambda b, f, be: (b, 0)),
    )
    return pl.pallas_call(
        functools.partial(_ffn_body, nf),
        grid_spec=grid_spec,
        out_shape=jax.ShapeDtypeStruct((ns, d), jnp.float32),
        compiler_params=pltpu.CompilerParams(
            dimension_semantics=("arbitrary", "arbitrary")),
        interpret=_INTERPRET,
    )(be, xs, W1.astype(jnp.bfloat16), b1.reshape(e, 1, f_dim),
      W2.astype(jnp.bfloat16), b2.reshape(e, 1, d), gw)


def _sc_dispatch(x_flat, slot0, slot1, w0, w1, ns):
    n, d = x_flat.shape
    info = plsc.get_sparse_core_info()
    nw = info.num_cores * info.num_subcores
    bw = n // nw
    chunk = 64
    steps = bw // chunk
    mesh = plsc.VectorSubcoreMesh(core_axis_name="c", subcore_axis_name="s")

    @functools.partial(
        pl.kernel,
        out_type=(
            jax.ShapeDtypeStruct((ns, d), jnp.float32),
            jax.ShapeDtypeStruct((ns,), jnp.float32),
        ),
        mesh=mesh,
        scratch_types=[
            pltpu.VMEM((chunk, d), jnp.float32),
            pltpu.VMEM((chunk,), jnp.int32),
            pltpu.VMEM((chunk,), jnp.int32),
            pltpu.VMEM((chunk,), jnp.float32),
            pltpu.VMEM((chunk,), jnp.float32),
            pltpu.SemaphoreType.DMA,
            pltpu.SemaphoreType.DMA,
            pltpu.SemaphoreType.DMA,
            pltpu.SemaphoreType.DMA,
        ],
    )
    def k(x_hbm, s0_hbm, s1_hbm, w0_hbm, w1_hbm, xs_hbm, gw_hbm,
          rows_v, i0_v, i1_v, w0_v, w1_v, sem0, sem1, semw0, semw1):
        wid = lax.axis_index("s") * info.num_cores + lax.axis_index("c")
        base = wid * bw

        @pl.loop(0, steps)
        def _(ci):
            off = base + ci * chunk
            pltpu.sync_copy(x_hbm.at[pl.ds(off, chunk)], rows_v)
            pltpu.sync_copy(s0_hbm.at[pl.ds(off, chunk)], i0_v)
            pltpu.sync_copy(s1_hbm.at[pl.ds(off, chunk)], i1_v)
            pltpu.sync_copy(w0_hbm.at[pl.ds(off, chunk)], w0_v)
            pltpu.sync_copy(w1_hbm.at[pl.ds(off, chunk)], w1_v)
            c0 = pltpu.async_copy(rows_v, xs_hbm.at[i0_v], sem0)
            c1 = pltpu.async_copy(rows_v, xs_hbm.at[i1_v], sem1)
            cw0 = pltpu.async_copy(w0_v, gw_hbm.at[i0_v], semw0)
            cw1 = pltpu.async_copy(w1_v, gw_hbm.at[i1_v], semw1)
            c0.wait()
            c1.wait()
            cw0.wait()
            cw1.wait()

    return k(x_flat, slot0, slot1, w0, w1)


def _sc_combine_gather(ys, slot0, slot1):
    n = slot0.shape[0]
    d = ys.shape[1]
    info = plsc.get_sparse_core_info()
    nw = info.num_cores * info.num_subcores
    bw = n // nw
    chunk = 32
    steps = bw // chunk
    mesh = plsc.VectorSubcoreMesh(core_axis_name="c", subcore_axis_name="s")

    @functools.partial(
        pl.kernel,
        out_type=(
            jax.ShapeDtypeStruct((n, d), jnp.float32),
            jax.ShapeDtypeStruct((n, d), jnp.float32),
        ),
        mesh=mesh,
        scratch_types=[
            pltpu.VMEM((chunk,), jnp.int32),
            pltpu.VMEM((chunk,), jnp.int32),
            pltpu.VMEM((chunk, d), jnp.float32),
            pltpu.VMEM((chunk, d), jnp.float32),
            pltpu.SemaphoreType.DMA,
            pltpu.SemaphoreType.DMA,
        ],
    )
    def k(ys_hbm, s0_hbm, s1_hbm, g0_hbm, g1_hbm, i0_v, i1_v, r0_v, r1_v,
          sem0, sem1):
        wid = lax.axis_index("s") * info.num_cores + lax.axis_index("c")
        base = wid * bw

        @pl.loop(0, steps)
        def _(ci):
            off = base + ci * chunk
            pltpu.sync_copy(s0_hbm.at[pl.ds(off, chunk)], i0_v)
            pltpu.sync_copy(s1_hbm.at[pl.ds(off, chunk)], i1_v)
            c0 = pltpu.async_copy(ys_hbm.at[i0_v], r0_v, sem0)
            c1 = pltpu.async_copy(ys_hbm.at[i1_v], r1_v, sem1)
            c0.wait()
            c1.wait()
            pltpu.sync_copy(r0_v, g0_hbm.at[pl.ds(off, chunk)])
            pltpu.sync_copy(r1_v, g1_hbm.at[pl.ds(off, chunk)])

    return k(ys, slot0, slot1)


def _add_body(a_ref, b_ref, o_ref):
    o_ref[...] = a_ref[...] + b_ref[...]


def _tc_add(a, b):
    n, d = a.shape
    rt = 512
    return pl.pallas_call(
        _add_body,
        grid=(n // rt,),
        in_specs=[
            pl.BlockSpec((rt, d), lambda i: (i, 0)),
            pl.BlockSpec((rt, d), lambda i: (i, 0)),
        ],
        out_specs=pl.BlockSpec((rt, d), lambda i: (i, 0)),
        out_shape=jax.ShapeDtypeStruct((n, d), jnp.float32),
        interpret=_INTERPRET,
    )(a, b)


def kernel(x, Wg, bg, W1, b1, W2, b2):
    b, s, d = x.shape
    e = Wg.shape[1]
    n = b * s
    nk = n * _TOPK
    ns = nk + e * _T
    x_flat = x.reshape(n, d)

    eidx, wgt, counts = _router(x_flat, Wg, bg)

    padded = ((counts + (_T - 1)) // _T) * _T
    ends = jnp.cumsum(padded, axis=1)
    seg = (ends - padded).astype(jnp.float32)
    slot2 = _slot_assign(eidx, seg)
    slot0 = slot2[:, 0]
    slot1 = slot2[:, 1]
    w0 = wgt[:, 0]
    w1 = wgt[:, 1]
    blk_start = jnp.arange(ns // _T, dtype=jnp.int32) * _T
    be = jnp.minimum(
        jnp.sum((blk_start[:, None] >= ends[0, :e][None, :]).astype(
            jnp.int32), axis=1), e - 1).astype(jnp.int32)

    xs, gw = _sc_dispatch(x_flat, slot0, slot1, w0, w1, ns)
    ys = _ffn(xs, W1, b1, W2, b2, gw.reshape(ns, 1), be)
    g0, g1 = _sc_combine_gather(ys, slot0, slot1)
    y = _tc_add(g0, g1)
    return y.reshape(b, s, d)


# speedup vs baseline: 1.0129x; 1.0129x over previous
import functools

import jax
import jax.numpy as jnp
from jax import lax
from jax.experimental import pallas as pl
from jax.experimental.pallas import tpu as pltpu
from jax.experimental.pallas import tpu_sc as plsc

_TOPK = 2
_T = 256
_FT = 1024
_RT = 512
_INTERPRET = False


def _router_body(x_ref, wg_ref, bg_ref, eidx_ref, wgt_ref, cnt_ref):
    step = pl.program_id(0)
    x = x_ref[...]
    logits = jnp.dot(x, wg_ref[...], preferred_element_type=jnp.float32)
    logits = logits + bg_ref[...]
    m = jnp.max(logits, axis=1, keepdims=True)
    ex = jnp.exp(logits - m)
    probs = ex / jnp.sum(ex, axis=1, keepdims=True)
    lane = jax.lax.broadcasted_iota(jnp.int32, probs.shape, 1)
    big = jnp.int32(10**6)
    m0 = jnp.max(probs, axis=1, keepdims=True)
    i0 = jnp.min(jnp.where(probs == m0, lane, big), axis=1, keepdims=True)
    probs1 = jnp.where(lane == i0, -1.0, probs)
    m1 = jnp.max(probs1, axis=1, keepdims=True)
    i1 = jnp.min(jnp.where(probs1 == m1, lane, big), axis=1, keepdims=True)
    eidx_ref[...] = jnp.where(lane == 0, i0, jnp.where(lane == 1, i1, 0))
    wgt_ref[...] = jnp.where(lane == 0, m0, jnp.where(lane == 1, m1, 0.0))
    oh = ((lane == i0) | (lane == i1)).astype(jnp.int32)
    chunk_counts = jnp.sum(oh, axis=0, keepdims=True)

    @pl.when(step == 0)
    def _():
        cnt_ref[...] = chunk_counts

    @pl.when(step > 0)
    def _():
        cnt_ref[...] = cnt_ref[...] + chunk_counts


def _router(x_flat, Wg, bg):
    n, d = x_flat.shape
    e = Wg.shape[1]
    wg_pad = jnp.zeros((d, 128), jnp.float32).at[:, :e].set(Wg)
    bg_pad = jnp.full((1, 128), -1e30, jnp.float32).at[0, :e].set(bg)
    return pl.pallas_call(
        _router_body,
        grid=(n // _RT,),
        in_specs=[
            pl.BlockSpec((_RT, d), lambda i: (i, 0)),
            pl.BlockSpec((d, 128), lambda i: (0, 0)),
            pl.BlockSpec((1, 128), lambda i: (0, 0)),
        ],
        out_specs=[
            pl.BlockSpec((_RT, 128), lambda i: (i, 0)),
            pl.BlockSpec((_RT, 128), lambda i: (i, 0)),
            pl.BlockSpec((1, 128), lambda i: (0, 0)),
        ],
        out_shape=[
            jax.ShapeDtypeStruct((n, 128), jnp.int32),
            jax.ShapeDtypeStruct((n, 128), jnp.float32),
            jax.ShapeDtypeStruct((1, 128), jnp.int32),
        ],
        compiler_params=pltpu.CompilerParams(
            dimension_semantics=("arbitrary",)),
        interpret=_INTERPRET,
    )(x_flat, wg_pad, bg_pad)


def _slot_body(eidx_ref, seg_ref, slot_ref, run_ref):
    step = pl.program_id(0)

    @pl.when(step == 0)
    def _():
        run_ref[...] = jnp.zeros_like(run_ref)

    eidx = eidx_ref[...]
    lane = jax.lax.broadcasted_iota(jnp.int32, eidx.shape, 1)
    i0 = jnp.sum(jnp.where(lane == 0, eidx, 0), axis=1, keepdims=True)
    i1 = jnp.sum(jnp.where(lane == 1, eidx, 0), axis=1, keepdims=True)
    oh0 = (lane == i0).astype(jnp.float32)
    oh1 = (lane == i1).astype(jnp.float32)
    oh = jnp.concatenate([oh0, oh1], axis=0)
    m = 2 * _RT
    r = jax.lax.broadcasted_iota(jnp.int32, (m, m), 0)
    c = jax.lax.broadcasted_iota(jnp.int32, (m, m), 1)
    ltri = (r > c).astype(jnp.float32)
    pref = jnp.dot(ltri, oh, preferred_element_type=jnp.float32)
    base = run_ref[...] + seg_ref[...]
    slot_pe = jnp.sum(oh * (pref + base), axis=1, keepdims=True)
    s0 = slot_pe[:_RT]
    s1 = slot_pe[_RT:]
    slot_ref[...] = jnp.where(
        lane == 0, s0, jnp.where(lane == 1, s1, 0.0)).astype(jnp.int32)
    run_ref[...] = run_ref[...] + jnp.sum(oh, axis=0, keepdims=True)


def _slot_assign(eidx, seg):
    n = eidx.shape[0]
    return pl.pallas_call(
        _slot_body,
        grid=(n // _RT,),
        in_specs=[
            pl.BlockSpec((_RT, 128), lambda i: (i, 0)),
            pl.BlockSpec((1, 128), lambda i: (0, 0)),
        ],
        out_specs=pl.BlockSpec((_RT, 128), lambda i: (i, 0)),
        out_shape=jax.ShapeDtypeStruct((n, 128), jnp.int32),
        scratch_shapes=[pltpu.VMEM((1, 128), jnp.float32)],
        compiler_params=pltpu.CompilerParams(
            dimension_semantics=("arbitrary",)),
        interpret=_INTERPRET,
    )(eidx, seg)


def _erf(z):
    s = jnp.sign(z)
    a = jnp.abs(z)
    t = 1.0 / (1.0 + 0.3275911 * a)
    poly = t * (0.254829592 + t * (-0.284496736 + t * (1.421413741
           + t * (-1.453152027 + t * 1.061405429))))
    return s * (1.0 - poly * jnp.exp(-a * a))


def _gelu(h):
    return 0.5 * h * (1.0 + _erf(h * 0.7071067811865476))


def _ffn_body(nf, be_ref, xs_ref, w1_ref, b1_ref, w2_ref, b2_ref, gw_ref,
              out_ref):
    f = pl.program_id(1)
    xb = xs_ref[...].astype(jnp.bfloat16)
    h = jnp.dot(xb, w1_ref[0], preferred_element_type=jnp.float32)
    h = _gelu(h + b1_ref[0])
    acc = jnp.dot(h.astype(jnp.bfloat16), w2_ref[0],
                  preferred_element_type=jnp.float32)

    @pl.when(f == 0)
    def _():
        out_ref[...] = acc

    @pl.when(f > 0)
    def _():
        out_ref[...] = out_ref[...] + acc

    @pl.when(f == nf - 1)
    def _():
        out_ref[...] = (out_ref[...] + b2_ref[0]) * gw_ref[...]


def _ffn(xs, W1, b1, W2, b2, gw, be):
    ns, d = xs.shape
    e, _, f_dim = W1.shape
    nb = ns // _T
    nf = f_dim // _FT
    def _feff(b, f):
        return jnp.where(b % 2 == 0, f, nf - 1 - f)

    grid_spec = pltpu.PrefetchScalarGridSpec(
        num_scalar_prefetch=1,
        grid=(nb, nf),
        in_specs=[
            pl.BlockSpec((_T, d), lambda b, f, be: (b, 0)),
            pl.BlockSpec((1, d, _FT), lambda b, f, be: (be[b], 0, _feff(b, f))),
            pl.BlockSpec((1, 1, _FT), lambda b, f, be: (be[b], 0, _feff(b, f))),
            pl.BlockSpec((1, _FT, d), lambda b, f, be: (be[b], _feff(b, f), 0)),
            pl.BlockSpec((1, 1, d), lambda b, f, be: (be[b], 0, 0)),
            pl.BlockSpec((_T, 1), lambda b, f, be: (b, 0)),
        ],
        out_specs=pl.BlockSpec((_T, d), lambda b, f, be: (b, 0)),
    )
    return pl.pallas_call(
        functools.partial(_ffn_body, nf),
        grid_spec=grid_spec,
        out_shape=jax.ShapeDtypeStruct((ns, d), jnp.float32),
        compiler_params=pltpu.CompilerParams(
            dimension_semantics=("arbitrary", "arbitrary")),
        interpret=_INTERPRET,
    )(be, xs, W1.astype(jnp.bfloat16), b1.reshape(e, 1, f_dim),
      W2.astype(jnp.bfloat16), b2.reshape(e, 1, d), gw)


def _sc_dispatch(x_flat, slot0, slot1, w0, w1, ns):
    n, d = x_flat.shape
    info = plsc.get_sparse_core_info()
    nw = info.num_cores * info.num_subcores
    bw = n // nw
    chunk = 64
    steps = bw // chunk
    mesh = plsc.VectorSubcoreMesh(core_axis_name="c", subcore_axis_name="s")

    @functools.partial(
        pl.kernel,
        out_type=(
            jax.ShapeDtypeStruct((ns, d), jnp.float32),
            jax.ShapeDtypeStruct((ns,), jnp.float32),
        ),
        mesh=mesh,
        scratch_types=[
            pltpu.VMEM((chunk, d), jnp.float32),
            pltpu.VMEM((chunk,), jnp.int32),
            pltpu.VMEM((chunk,), jnp.int32),
            pltpu.VMEM((chunk,), jnp.float32),
            pltpu.VMEM((chunk,), jnp.float32),
            pltpu.SemaphoreType.DMA,
            pltpu.SemaphoreType.DMA,
            pltpu.SemaphoreType.DMA,
            pltpu.SemaphoreType.DMA,
        ],
    )
    def k(x_hbm, s0_hbm, s1_hbm, w0_hbm, w1_hbm, xs_hbm, gw_hbm,
          rows_v, i0_v, i1_v, w0_v, w1_v, sem0, sem1, semw0, semw1):
        wid = lax.axis_index("s") * info.num_cores + lax.axis_index("c")
        base = wid * bw

        @pl.loop(0, steps)
        def _(ci):
            off = base + ci * chunk
            pltpu.sync_copy(x_hbm.at[pl.ds(off, chunk)], rows_v)
            pltpu.sync_copy(s0_hbm.at[pl.ds(off, chunk)], i0_v)
            pltpu.sync_copy(s1_hbm.at[pl.ds(off, chunk)], i1_v)
            pltpu.sync_copy(w0_hbm.at[pl.ds(off, chunk)], w0_v)
            pltpu.sync_copy(w1_hbm.at[pl.ds(off, chunk)], w1_v)
            c0 = pltpu.async_copy(rows_v, xs_hbm.at[i0_v], sem0)
            c1 = pltpu.async_copy(rows_v, xs_hbm.at[i1_v], sem1)
            cw0 = pltpu.async_copy(w0_v, gw_hbm.at[i0_v], semw0)
            cw1 = pltpu.async_copy(w1_v, gw_hbm.at[i1_v], semw1)
            c0.wait()
            c1.wait()
            cw0.wait()
            cw1.wait()

    return k(x_flat, slot0, slot1, w0, w1)


def _sc_combine(ys, slot0, slot1):
    n = slot0.shape[0]
    d = ys.shape[1]
    info = plsc.get_sparse_core_info()
    nw = info.num_cores * info.num_subcores
    bw = n // nw
    chunk = 32
    steps = bw // chunk
    mesh = plsc.VectorSubcoreMesh(core_axis_name="c", subcore_axis_name="s")

    @functools.partial(
        pl.kernel,
        out_type=jax.ShapeDtypeStruct((n, d), jnp.float32),
        mesh=mesh,
        scratch_types=[
            pltpu.VMEM((chunk,), jnp.int32),
            pltpu.VMEM((chunk,), jnp.int32),
            pltpu.VMEM((chunk, d), jnp.float32),
            pltpu.VMEM((chunk, d), jnp.float32),
            pltpu.SemaphoreType.DMA,
            pltpu.SemaphoreType.DMA,
        ],
    )
    def k(ys_hbm, s0_hbm, s1_hbm, out_hbm, i0_v, i1_v, r0_v, r1_v, sem0,
          sem1):
        wid = lax.axis_index("s") * info.num_cores + lax.axis_index("c")
        base = wid * bw

        @pl.loop(0, steps)
        def _(ci):
            off = base + ci * chunk
            pltpu.sync_copy(s0_hbm.at[pl.ds(off, chunk)], i0_v)
            pltpu.sync_copy(s1_hbm.at[pl.ds(off, chunk)], i1_v)
            c0 = pltpu.async_copy(ys_hbm.at[i0_v], r0_v, sem0)
            c1 = pltpu.async_copy(ys_hbm.at[i1_v], r1_v, sem1)
            c0.wait()
            c1.wait()

            @pl.loop(0, chunk)
            def _(r):
                for j in range(d // 16):
                    sl = (r, pl.ds(j * 16, 16))
                    r0_v[sl] = r0_v[sl] + r1_v[sl]

            pltpu.sync_copy(r0_v, out_hbm.at[pl.ds(off, chunk)])

    return k(ys, slot0, slot1)


def kernel(x, Wg, bg, W1, b1, W2, b2):
    b, s, d = x.shape
    e = Wg.shape[1]
    n = b * s
    nk = n * _TOPK
    ns = nk + e * _T
    x_flat = x.reshape(n, d)

    eidx, wgt, counts = _router(x_flat, Wg, bg)

    padded = ((counts + (_T - 1)) // _T) * _T
    ends = jnp.cumsum(padded, axis=1)
    seg = (ends - padded).astype(jnp.float32)
    slot2 = _slot_assign(eidx, seg)
    slot0 = slot2[:, 0]
    slot1 = slot2[:, 1]
    w0 = wgt[:, 0]
    w1 = wgt[:, 1]
    blk_start = jnp.arange(ns // _T, dtype=jnp.int32) * _T
    be = jnp.minimum(
        jnp.sum((blk_start[:, None] >= ends[0, :e][None, :]).astype(
            jnp.int32), axis=1), e - 1).astype(jnp.int32)

    xs, gw = _sc_dispatch(x_flat, slot0, slot1, w0, w1, ns)
    ys = _ffn(xs, W1, b1, W2, b2, gw.reshape(ns, 1), be)
    y = _sc_combine(ys, slot0, slot1)
    return y.reshape(b, s, d)


# speedup vs baseline: 1.1269x; 1.1125x over previous
import functools

import jax
import jax.numpy as jnp
from jax import lax
from jax.experimental import pallas as pl
from jax.experimental.pallas import tpu as pltpu
from jax.experimental.pallas import tpu_sc as plsc

_TOPK = 2
_T = 256
_FT = 1024
_RT = 512
_INTERPRET = False


def _router_body(x_ref, wg_ref, bg_ref, eidx_ref, wgt_ref, cnt_ref):
    step = pl.program_id(0)
    x = x_ref[...]
    logits = jnp.dot(x, wg_ref[...], preferred_element_type=jnp.float32)
    logits = logits + bg_ref[...]
    m = jnp.max(logits, axis=1, keepdims=True)
    ex = jnp.exp(logits - m)
    probs = ex / jnp.sum(ex, axis=1, keepdims=True)
    lane = jax.lax.broadcasted_iota(jnp.int32, probs.shape, 1)
    big = jnp.int32(10**6)
    m0 = jnp.max(probs, axis=1, keepdims=True)
    i0 = jnp.min(jnp.where(probs == m0, lane, big), axis=1, keepdims=True)
    probs1 = jnp.where(lane == i0, -1.0, probs)
    m1 = jnp.max(probs1, axis=1, keepdims=True)
    i1 = jnp.min(jnp.where(probs1 == m1, lane, big), axis=1, keepdims=True)
    eidx_ref[...] = jnp.where(lane == 0, i0, jnp.where(lane == 1, i1, 0))
    wgt_ref[...] = jnp.where(lane == 0, m0, jnp.where(lane == 1, m1, 0.0))
    oh = ((lane == i0) | (lane == i1)).astype(jnp.int32)
    chunk_counts = jnp.sum(oh, axis=0, keepdims=True)

    @pl.when(step == 0)
    def _():
        cnt_ref[...] = chunk_counts

    @pl.when(step > 0)
    def _():
        cnt_ref[...] = cnt_ref[...] + chunk_counts


def _router(x_flat, Wg, bg):
    n, d = x_flat.shape
    e = Wg.shape[1]
    wg_pad = jnp.zeros((d, 128), jnp.float32).at[:, :e].set(Wg)
    bg_pad = jnp.full((1, 128), -1e30, jnp.float32).at[0, :e].set(bg)
    return pl.pallas_call(
        _router_body,
        grid=(n // _RT,),
        in_specs=[
            pl.BlockSpec((_RT, d), lambda i: (i, 0)),
            pl.BlockSpec((d, 128), lambda i: (0, 0)),
            pl.BlockSpec((1, 128), lambda i: (0, 0)),
        ],
        out_specs=[
            pl.BlockSpec((_RT, 128), lambda i: (i, 0)),
            pl.BlockSpec((_RT, 128), lambda i: (i, 0)),
            pl.BlockSpec((1, 128), lambda i: (0, 0)),
        ],
        out_shape=[
            jax.ShapeDtypeStruct((n, 128), jnp.int32),
            jax.ShapeDtypeStruct((n, 128), jnp.float32),
            jax.ShapeDtypeStruct((1, 128), jnp.int32),
        ],
        compiler_params=pltpu.CompilerParams(
            dimension_semantics=("arbitrary",)),
        interpret=_INTERPRET,
    )(x_flat, wg_pad, bg_pad)


def _slot_body(eidx_ref, seg_ref, slot_ref, run_ref):
    step = pl.program_id(0)

    @pl.when(step == 0)
    def _():
        run_ref[...] = jnp.zeros_like(run_ref)

    eidx = eidx_ref[...]
    lane = jax.lax.broadcasted_iota(jnp.int32, eidx.shape, 1)
    i0 = jnp.sum(jnp.where(lane == 0, eidx, 0), axis=1, keepdims=True)
    i1 = jnp.sum(jnp.where(lane == 1, eidx, 0), axis=1, keepdims=True)
    oh0 = (lane == i0).astype(jnp.float32)
    oh1 = (lane == i1).astype(jnp.float32)
    oh = jnp.concatenate([oh0, oh1], axis=0)
    m = 2 * _RT
    r = jax.lax.broadcasted_iota(jnp.int32, (m, m), 0)
    c = jax.lax.broadcasted_iota(jnp.int32, (m, m), 1)
    ltri = (r > c).astype(jnp.float32)
    pref = jnp.dot(ltri, oh, preferred_element_type=jnp.float32)
    base = run_ref[...] + seg_ref[...]
    slot_pe = jnp.sum(oh * (pref + base), axis=1, keepdims=True)
    s0 = slot_pe[:_RT]
    s1 = slot_pe[_RT:]
    slot_ref[...] = jnp.where(
        lane == 0, s0, jnp.where(lane == 1, s1, 0.0)).astype(jnp.int32)
    run_ref[...] = run_ref[...] + jnp.sum(oh, axis=0, keepdims=True)


def _slot_assign(eidx, seg):
    n = eidx.shape[0]
    return pl.pallas_call(
        _slot_body,
        grid=(n // _RT,),
        in_specs=[
            pl.BlockSpec((_RT, 128), lambda i: (i, 0)),
            pl.BlockSpec((1, 128), lambda i: (0, 0)),
        ],
        out_specs=pl.BlockSpec((_RT, 128), lambda i: (i, 0)),
        out_shape=jax.ShapeDtypeStruct((n, 128), jnp.int32),
        scratch_shapes=[pltpu.VMEM((1, 128), jnp.float32)],
        compiler_params=pltpu.CompilerParams(
            dimension_semantics=("arbitrary",)),
        interpret=_INTERPRET,
    )(eidx, seg)


def _erf(z):
    s = jnp.sign(z)
    a = jnp.abs(z)
    t = 1.0 / (1.0 + 0.3275911 * a)
    poly = t * (0.254829592 + t * (-0.284496736 + t * (1.421413741
           + t * (-1.453152027 + t * 1.061405429))))
    return s * (1.0 - poly * jnp.exp(-a * a))


def _gelu(h):
    return 0.5 * h * (1.0 + _erf(h * 0.7071067811865476))


_FTA = 2048
_DTB = 512


def _ffn1_body(nfa, be_ref, xs_ref, w1_ref, b1_ref, h_ref, w1b_ref,
               last_ref):
    fa = pl.program_id(0)
    b = pl.program_id(1)

    @pl.when((fa == 0) & (b == 0))
    def _():
        last_ref[0] = -1

    key = be_ref[b] * nfa + fa

    @pl.when(key != last_ref[0])
    def _():
        w1b_ref[...] = w1_ref[0].astype(jnp.bfloat16)
        last_ref[0] = key

    xb = xs_ref[...].astype(jnp.bfloat16)
    hh = jnp.dot(xb, w1b_ref[...], preferred_element_type=jnp.float32)
    h_ref[...] = _gelu(hh + b1_ref[0]).astype(jnp.bfloat16)


def _ffn2_body(nd, be_ref, h_ref, w2_ref, b2_ref, gw_ref, out_ref, w2b_ref,
               last_ref):
    dd = pl.program_id(0)
    b = pl.program_id(1)

    @pl.when((dd == 0) & (b == 0))
    def _():
        last_ref[0] = -1

    key = be_ref[b] * nd + dd

    @pl.when(key != last_ref[0])
    def _():
        w2b_ref[...] = w2_ref[0].astype(jnp.bfloat16)
        last_ref[0] = key

    acc = jnp.dot(h_ref[...], w2b_ref[...],
                  preferred_element_type=jnp.float32)
    out_ref[...] = (acc + b2_ref[0]) * gw_ref[...]


def _ffn(xs, W1, b1, W2, b2, gw, be):
    ns, d = xs.shape
    e, _, f_dim = W1.shape
    nb = ns // _T
    nfa = f_dim // _FTA
    nd = d // _DTB

    spec1 = pltpu.PrefetchScalarGridSpec(
        num_scalar_prefetch=1,
        grid=(nfa, nb),
        in_specs=[
            pl.BlockSpec((_T, d), lambda fa, b, be: (b, 0)),
            pl.BlockSpec((1, d, _FTA), lambda fa, b, be: (be[b], 0, fa)),
            pl.BlockSpec((1, 1, _FTA), lambda fa, b, be: (be[b], 0, fa)),
        ],
        out_specs=pl.BlockSpec((_T, _FTA), lambda fa, b, be: (b, fa)),
        scratch_shapes=[
            pltpu.VMEM((d, _FTA), jnp.bfloat16),
            pltpu.SMEM((1,), jnp.int32),
        ],
    )
    h = pl.pallas_call(
        functools.partial(_ffn1_body, nfa),
        grid_spec=spec1,
        out_shape=jax.ShapeDtypeStruct((ns, f_dim), jnp.bfloat16),
        compiler_params=pltpu.CompilerParams(
            dimension_semantics=("arbitrary", "arbitrary")),
        interpret=_INTERPRET,
    )(be, xs, W1, b1.reshape(e, 1, f_dim))

    spec2 = pltpu.PrefetchScalarGridSpec(
        num_scalar_prefetch=1,
        grid=(nd, nb),
        in_specs=[
            pl.BlockSpec((_T, f_dim), lambda dd, b, be: (b, 0)),
            pl.BlockSpec((1, f_dim, _DTB), lambda dd, b, be: (be[b], 0, dd)),
            pl.BlockSpec((1, 1, _DTB), lambda dd, b, be: (be[b], 0, dd)),
            pl.BlockSpec((_T, 1), lambda dd, b, be: (b, 0)),
        ],
        out_specs=pl.BlockSpec((_T, _DTB), lambda dd, b, be: (b, dd)),
        scratch_shapes=[
            pltpu.VMEM((f_dim, _DTB), jnp.bfloat16),
            pltpu.SMEM((1,), jnp.int32),
        ],
    )
    return pl.pallas_call(
        functools.partial(_ffn2_body, nd),
        grid_spec=spec2,
        out_shape=jax.ShapeDtypeStruct((ns, d), jnp.float32),
        compiler_params=pltpu.CompilerParams(
            dimension_semantics=("arbitrary", "arbitrary")),
        interpret=_INTERPRET,
    )(be, h, W2, b2.reshape(e, 1, d), gw)


def _sc_dispatch(x_flat, slot0, slot1, w0, w1, ns):
    n, d = x_flat.shape
    info = plsc.get_sparse_core_info()
    nw = info.num_cores * info.num_subcores
    bw = n // nw
    chunk = 64
    steps = bw // chunk
    mesh = plsc.VectorSubcoreMesh(core_axis_name="c", subcore_axis_name="s")

    @functools.partial(
        pl.kernel,
        out_type=(
            jax.ShapeDtypeStruct((ns, d), jnp.float32),
            jax.ShapeDtypeStruct((ns,), jnp.float32),
        ),
        mesh=mesh,
        scratch_types=[
            pltpu.VMEM((chunk, d), jnp.float32),
            pltpu.VMEM((chunk,), jnp.int32),
            pltpu.VMEM((chunk,), jnp.int32),
            pltpu.VMEM((chunk,), jnp.float32),
            pltpu.VMEM((chunk,), jnp.float32),
            pltpu.SemaphoreType.DMA,
            pltpu.SemaphoreType.DMA,
            pltpu.SemaphoreType.DMA,
            pltpu.SemaphoreType.DMA,
        ],
    )
    def k(x_hbm, s0_hbm, s1_hbm, w0_hbm, w1_hbm, xs_hbm, gw_hbm,
          rows_v, i0_v, i1_v, w0_v, w1_v, sem0, sem1, semw0, semw1):
        wid = lax.axis_index("s") * info.num_cores + lax.axis_index("c")
        base = wid * bw

        @pl.loop(0, steps)
        def _(ci):
            off = base + ci * chunk
            pltpu.sync_copy(x_hbm.at[pl.ds(off, chunk)], rows_v)
            pltpu.sync_copy(s0_hbm.at[pl.ds(off, chunk)], i0_v)
            pltpu.sync_copy(s1_hbm.at[pl.ds(off, chunk)], i1_v)
            pltpu.sync_copy(w0_hbm.at[pl.ds(off, chunk)], w0_v)
            pltpu.sync_copy(w1_hbm.at[pl.ds(off, chunk)], w1_v)
            c0 = pltpu.async_copy(rows_v, xs_hbm.at[i0_v], sem0)
            c1 = pltpu.async_copy(rows_v, xs_hbm.at[i1_v], sem1)
            cw0 = pltpu.async_copy(w0_v, gw_hbm.at[i0_v], semw0)
            cw1 = pltpu.async_copy(w1_v, gw_hbm.at[i1_v], semw1)
            c0.wait()
            c1.wait()
            cw0.wait()
            cw1.wait()

    return k(x_flat, slot0, slot1, w0, w1)


def _sc_combine(ys, slot0, slot1):
    n = slot0.shape[0]
    d = ys.shape[1]
    info = plsc.get_sparse_core_info()
    nw = info.num_cores * info.num_subcores
    bw = n // nw
    chunk = 32
    steps = bw // chunk
    mesh = plsc.VectorSubcoreMesh(core_axis_name="c", subcore_axis_name="s")

    @functools.partial(
        pl.kernel,
        out_type=jax.ShapeDtypeStruct((n, d), jnp.float32),
        mesh=mesh,
        scratch_types=[
            pltpu.VMEM((chunk,), jnp.int32),
            pltpu.VMEM((chunk,), jnp.int32),
            pltpu.VMEM((chunk, d), jnp.float32),
            pltpu.VMEM((chunk, d), jnp.float32),
            pltpu.SemaphoreType.DMA,
            pltpu.SemaphoreType.DMA,
        ],
    )
    def k(ys_hbm, s0_hbm, s1_hbm, out_hbm, i0_v, i1_v, r0_v, r1_v, sem0,
          sem1):
        wid = lax.axis_index("s") * info.num_cores + lax.axis_index("c")
        base = wid * bw

        @pl.loop(0, steps)
        def _(ci):
            off = base + ci * chunk
            pltpu.sync_copy(s0_hbm.at[pl.ds(off, chunk)], i0_v)
            pltpu.sync_copy(s1_hbm.at[pl.ds(off, chunk)], i1_v)
            c0 = pltpu.async_copy(ys_hbm.at[i0_v], r0_v, sem0)
            c1 = pltpu.async_copy(ys_hbm.at[i1_v], r1_v, sem1)
            c0.wait()
            c1.wait()

            @pl.loop(0, chunk)
            def _(r):
                for j in range(d // 16):
                    sl = (r, pl.ds(j * 16, 16))
                    r0_v[sl] = r0_v[sl] + r1_v[sl]

            pltpu.sync_copy(r0_v, out_hbm.at[pl.ds(off, chunk)])

    return k(ys, slot0, slot1)


def kernel(x, Wg, bg, W1, b1, W2, b2):
    b, s, d = x.shape
    e = Wg.shape[1]
    n = b * s
    nk = n * _TOPK
    ns = nk + e * _T
    x_flat = x.reshape(n, d)

    eidx, wgt, counts = _router(x_flat, Wg, bg)

    padded = ((counts + (_T - 1)) // _T) * _T
    ends = jnp.cumsum(padded, axis=1)
    seg = (ends - padded).astype(jnp.float32)
    slot2 = _slot_assign(eidx, seg)
    slot0 = slot2[:, 0]
    slot1 = slot2[:, 1]
    w0 = wgt[:, 0]
    w1 = wgt[:, 1]
    blk_start = jnp.arange(ns // _T, dtype=jnp.int32) * _T
    be = jnp.minimum(
        jnp.sum((blk_start[:, None] >= ends[0, :e][None, :]).astype(
            jnp.int32), axis=1), e - 1).astype(jnp.int32)

    xs, gw = _sc_dispatch(x_flat, slot0, slot1, w0, w1, ns)
    ys = _ffn(xs, W1, b1, W2, b2, gw.reshape(ns, 1), be)
    y = _sc_combine(ys, slot0, slot1)
    return y.reshape(b, s, d)


# speedup vs baseline: 1.2149x; 1.0781x over previous
import functools

import jax
import jax.numpy as jnp
from jax import lax
from jax.experimental import pallas as pl
from jax.experimental.pallas import tpu as pltpu
from jax.experimental.pallas import tpu_sc as plsc

_TOPK = 2
_T = 256
_FT = 1024
_RT = 512
_INTERPRET = False


def _router_body(x_ref, wg_ref, bg_ref, eidx_ref, wgt_ref, cnt_ref):
    step = pl.program_id(0)
    x = x_ref[...]
    logits = jnp.dot(x, wg_ref[...], preferred_element_type=jnp.float32)
    logits = logits + bg_ref[...]
    m = jnp.max(logits, axis=1, keepdims=True)
    ex = jnp.exp(logits - m)
    probs = ex / jnp.sum(ex, axis=1, keepdims=True)
    lane = jax.lax.broadcasted_iota(jnp.int32, probs.shape, 1)
    big = jnp.int32(10**6)
    m0 = jnp.max(probs, axis=1, keepdims=True)
    i0 = jnp.min(jnp.where(probs == m0, lane, big), axis=1, keepdims=True)
    probs1 = jnp.where(lane == i0, -1.0, probs)
    m1 = jnp.max(probs1, axis=1, keepdims=True)
    i1 = jnp.min(jnp.where(probs1 == m1, lane, big), axis=1, keepdims=True)
    eidx_ref[...] = jnp.where(lane == 0, i0, jnp.where(lane == 1, i1, 0))
    wgt_ref[...] = jnp.where(lane == 0, m0, jnp.where(lane == 1, m1, 0.0))
    oh = ((lane == i0) | (lane == i1)).astype(jnp.int32)
    chunk_counts = jnp.sum(oh, axis=0, keepdims=True)

    @pl.when(step == 0)
    def _():
        cnt_ref[...] = chunk_counts

    @pl.when(step > 0)
    def _():
        cnt_ref[...] = cnt_ref[...] + chunk_counts


def _router(x_flat, Wg, bg):
    n, d = x_flat.shape
    e = Wg.shape[1]
    wg_pad = jnp.zeros((d, 128), jnp.float32).at[:, :e].set(Wg)
    bg_pad = jnp.full((1, 128), -1e30, jnp.float32).at[0, :e].set(bg)
    return pl.pallas_call(
        _router_body,
        grid=(n // _RT,),
        in_specs=[
            pl.BlockSpec((_RT, d), lambda i: (i, 0)),
            pl.BlockSpec((d, 128), lambda i: (0, 0)),
            pl.BlockSpec((1, 128), lambda i: (0, 0)),
        ],
        out_specs=[
            pl.BlockSpec((_RT, 128), lambda i: (i, 0)),
            pl.BlockSpec((_RT, 128), lambda i: (i, 0)),
            pl.BlockSpec((1, 128), lambda i: (0, 0)),
        ],
        out_shape=[
            jax.ShapeDtypeStruct((n, 128), jnp.int32),
            jax.ShapeDtypeStruct((n, 128), jnp.float32),
            jax.ShapeDtypeStruct((1, 128), jnp.int32),
        ],
        compiler_params=pltpu.CompilerParams(
            dimension_semantics=("arbitrary",)),
        interpret=_INTERPRET,
    )(x_flat, wg_pad, bg_pad)


def _slot_body(eidx_ref, seg_ref, slot_ref, run_ref):
    step = pl.program_id(0)

    @pl.when(step == 0)
    def _():
        run_ref[...] = jnp.zeros_like(run_ref)

    eidx = eidx_ref[...]
    lane = jax.lax.broadcasted_iota(jnp.int32, eidx.shape, 1)
    i0 = jnp.sum(jnp.where(lane == 0, eidx, 0), axis=1, keepdims=True)
    i1 = jnp.sum(jnp.where(lane == 1, eidx, 0), axis=1, keepdims=True)
    oh0 = (lane == i0).astype(jnp.float32)
    oh1 = (lane == i1).astype(jnp.float32)
    oh = jnp.concatenate([oh0, oh1], axis=0)
    m = 2 * _RT
    r = jax.lax.broadcasted_iota(jnp.int32, (m, m), 0)
    c = jax.lax.broadcasted_iota(jnp.int32, (m, m), 1)
    ltri = (r > c).astype(jnp.float32)
    pref = jnp.dot(ltri, oh, preferred_element_type=jnp.float32)
    base = run_ref[...] + seg_ref[...]
    slot_pe = jnp.sum(oh * (pref + base), axis=1, keepdims=True)
    s0 = slot_pe[:_RT]
    s1 = slot_pe[_RT:]
    slot_ref[...] = jnp.where(
        lane == 0, s0, jnp.where(lane == 1, s1, 0.0)).astype(jnp.int32)
    run_ref[...] = run_ref[...] + jnp.sum(oh, axis=0, keepdims=True)


def _slot_assign(eidx, seg):
    n = eidx.shape[0]
    return pl.pallas_call(
        _slot_body,
        grid=(n // _RT,),
        in_specs=[
            pl.BlockSpec((_RT, 128), lambda i: (i, 0)),
            pl.BlockSpec((1, 128), lambda i: (0, 0)),
        ],
        out_specs=pl.BlockSpec((_RT, 128), lambda i: (i, 0)),
        out_shape=jax.ShapeDtypeStruct((n, 128), jnp.int32),
        scratch_shapes=[pltpu.VMEM((1, 128), jnp.float32)],
        compiler_params=pltpu.CompilerParams(
            dimension_semantics=("arbitrary",)),
        interpret=_INTERPRET,
    )(eidx, seg)


def _erf(z):
    s = jnp.sign(z)
    a = jnp.abs(z)
    t = 1.0 / (1.0 + 0.3275911 * a)
    poly = t * (0.254829592 + t * (-0.284496736 + t * (1.421413741
           + t * (-1.453152027 + t * 1.061405429))))
    return s * (1.0 - poly * jnp.exp(-a * a))


def _gelu(h):
    return 0.5 * h * (1.0 + _erf(h * 0.7071067811865476))


_FTA = 4096
_DTB = 1024


def _ffn1_body(nfa, be_ref, xs_ref, w1_ref, b1_ref, h_ref, w1b_ref,
               last_ref):
    fa = pl.program_id(0)
    b = pl.program_id(1)

    @pl.when((fa == 0) & (b == 0))
    def _():
        last_ref[0] = -1

    key = be_ref[b] * nfa + fa

    @pl.when(key != last_ref[0])
    def _():
        w1b_ref[...] = w1_ref[0].astype(jnp.bfloat16)
        last_ref[0] = key

    xb = xs_ref[...].astype(jnp.bfloat16)
    hh = jnp.dot(xb, w1b_ref[...], preferred_element_type=jnp.float32)
    h_ref[...] = _gelu(hh + b1_ref[0]).astype(jnp.bfloat16)


def _ffn2_body(nd, be_ref, h_ref, w2_ref, b2_ref, gw_ref, out_ref, w2b_ref,
               last_ref):
    dd = pl.program_id(0)
    b = pl.program_id(1)

    @pl.when((dd == 0) & (b == 0))
    def _():
        last_ref[0] = -1

    key = be_ref[b] * nd + dd

    @pl.when(key != last_ref[0])
    def _():
        w2b_ref[...] = w2_ref[0].astype(jnp.bfloat16)
        last_ref[0] = key

    acc = jnp.dot(h_ref[...], w2b_ref[...],
                  preferred_element_type=jnp.float32)
    out_ref[...] = (acc + b2_ref[0]) * gw_ref[...]


def _ffn(xs, W1, b1, W2, b2, gw, be):
    ns, d = xs.shape
    e, _, f_dim = W1.shape
    nb = ns // _T
    nfa = f_dim // _FTA
    nd = d // _DTB

    spec1 = pltpu.PrefetchScalarGridSpec(
        num_scalar_prefetch=1,
        grid=(nfa, nb),
        in_specs=[
            pl.BlockSpec((_T, d), lambda fa, b, be: (b, 0)),
            pl.BlockSpec((1, d, _FTA), lambda fa, b, be: (be[b], 0, fa)),
            pl.BlockSpec((1, 1, _FTA), lambda fa, b, be: (be[b], 0, fa)),
        ],
        out_specs=pl.BlockSpec((_T, _FTA), lambda fa, b, be: (b, fa)),
        scratch_shapes=[
            pltpu.VMEM((d, _FTA), jnp.bfloat16),
            pltpu.SMEM((1,), jnp.int32),
        ],
    )
    h = pl.pallas_call(
        functools.partial(_ffn1_body, nfa),
        grid_spec=spec1,
        out_shape=jax.ShapeDtypeStruct((ns, f_dim), jnp.bfloat16),
        compiler_params=pltpu.CompilerParams(
            dimension_semantics=("arbitrary", "arbitrary")),
        interpret=_INTERPRET,
    )(be, xs, W1, b1.reshape(e, 1, f_dim))

    spec2 = pltpu.PrefetchScalarGridSpec(
        num_scalar_prefetch=1,
        grid=(nd, nb),
        in_specs=[
            pl.BlockSpec((_T, f_dim), lambda dd, b, be: (b, 0)),
            pl.BlockSpec((1, f_dim, _DTB), lambda dd, b, be: (be[b], 0, dd)),
            pl.BlockSpec((1, 1, _DTB), lambda dd, b, be: (be[b], 0, dd)),
            pl.BlockSpec((_T, 1), lambda dd, b, be: (b, 0)),
        ],
        out_specs=pl.BlockSpec((_T, _DTB), lambda dd, b, be: (b, dd)),
        scratch_shapes=[
            pltpu.VMEM((f_dim, _DTB), jnp.bfloat16),
            pltpu.SMEM((1,), jnp.int32),
        ],
    )
    return pl.pallas_call(
        functools.partial(_ffn2_body, nd),
        grid_spec=spec2,
        out_shape=jax.ShapeDtypeStruct((ns, d), jnp.float32),
        compiler_params=pltpu.CompilerParams(
            dimension_semantics=("arbitrary", "arbitrary")),
        interpret=_INTERPRET,
    )(be, h, W2, b2.reshape(e, 1, d), gw)


def _sc_dispatch(x_flat, slot0, slot1, w0, w1, ns):
    n, d = x_flat.shape
    info = plsc.get_sparse_core_info()
    nw = info.num_cores * info.num_subcores
    bw = n // nw
    chunk = 64
    steps = bw // chunk
    mesh = plsc.VectorSubcoreMesh(core_axis_name="c", subcore_axis_name="s")

    @functools.partial(
        pl.kernel,
        out_type=(
            jax.ShapeDtypeStruct((ns, d), jnp.float32),
            jax.ShapeDtypeStruct((ns,), jnp.float32),
        ),
        mesh=mesh,
        scratch_types=[
            pltpu.VMEM((chunk, d), jnp.float32),
            pltpu.VMEM((chunk,), jnp.int32),
            pltpu.VMEM((chunk,), jnp.int32),
            pltpu.VMEM((chunk,), jnp.float32),
            pltpu.VMEM((chunk,), jnp.float32),
            pltpu.SemaphoreType.DMA,
            pltpu.SemaphoreType.DMA,
            pltpu.SemaphoreType.DMA,
            pltpu.SemaphoreType.DMA,
        ],
    )
    def k(x_hbm, s0_hbm, s1_hbm, w0_hbm, w1_hbm, xs_hbm, gw_hbm,
          rows_v, i0_v, i1_v, w0_v, w1_v, sem0, sem1, semw0, semw1):
        wid = lax.axis_index("s") * info.num_cores + lax.axis_index("c")
        base = wid * bw

        @pl.loop(0, steps)
        def _(ci):
            off = base + ci * chunk
            pltpu.sync_copy(x_hbm.at[pl.ds(off, chunk)], rows_v)
            pltpu.sync_copy(s0_hbm.at[pl.ds(off, chunk)], i0_v)
            pltpu.sync_copy(s1_hbm.at[pl.ds(off, chunk)], i1_v)
            pltpu.sync_copy(w0_hbm.at[pl.ds(off, chunk)], w0_v)
            pltpu.sync_copy(w1_hbm.at[pl.ds(off, chunk)], w1_v)
            c0 = pltpu.async_copy(rows_v, xs_hbm.at[i0_v], sem0)
            c1 = pltpu.async_copy(rows_v, xs_hbm.at[i1_v], sem1)
            cw0 = pltpu.async_copy(w0_v, gw_hbm.at[i0_v], semw0)
            cw1 = pltpu.async_copy(w1_v, gw_hbm.at[i1_v], semw1)
            c0.wait()
            c1.wait()
            cw0.wait()
            cw1.wait()

    return k(x_flat, slot0, slot1, w0, w1)


def _sc_combine(ys, slot0, slot1):
    n = slot0.shape[0]
    d = ys.shape[1]
    info = plsc.get_sparse_core_info()
    nw = info.num_cores * info.num_subcores
    bw = n // nw
    chunk = 32
    steps = bw // chunk
    mesh = plsc.VectorSubcoreMesh(core_axis_name="c", subcore_axis_name="s")

    @functools.partial(
        pl.kernel,
        out_type=jax.ShapeDtypeStruct((n, d), jnp.float32),
        mesh=mesh,
        scratch_types=[
            pltpu.VMEM((chunk,), jnp.int32),
            pltpu.VMEM((chunk,), jnp.int32),
            pltpu.VMEM((chunk, d), jnp.float32),
            pltpu.VMEM((chunk, d), jnp.float32),
            pltpu.SemaphoreType.DMA,
            pltpu.SemaphoreType.DMA,
        ],
    )
    def k(ys_hbm, s0_hbm, s1_hbm, out_hbm, i0_v, i1_v, r0_v, r1_v, sem0,
          sem1):
        wid = lax.axis_index("s") * info.num_cores + lax.axis_index("c")
        base = wid * bw

        @pl.loop(0, steps)
        def _(ci):
            off = base + ci * chunk
            pltpu.sync_copy(s0_hbm.at[pl.ds(off, chunk)], i0_v)
            pltpu.sync_copy(s1_hbm.at[pl.ds(off, chunk)], i1_v)
            c0 = pltpu.async_copy(ys_hbm.at[i0_v], r0_v, sem0)
            c1 = pltpu.async_copy(ys_hbm.at[i1_v], r1_v, sem1)
            c0.wait()
            c1.wait()

            @pl.loop(0, chunk)
            def _(r):
                for j in range(d // 16):
                    sl = (r, pl.ds(j * 16, 16))
                    r0_v[sl] = r0_v[sl] + r1_v[sl]

            pltpu.sync_copy(r0_v, out_hbm.at[pl.ds(off, chunk)])

    return k(ys, slot0, slot1)


def kernel(x, Wg, bg, W1, b1, W2, b2):
    b, s, d = x.shape
    e = Wg.shape[1]
    n = b * s
    nk = n * _TOPK
    ns = nk + e * _T
    x_flat = x.reshape(n, d)

    eidx, wgt, counts = _router(x_flat, Wg, bg)

    padded = ((counts + (_T - 1)) // _T) * _T
    ends = jnp.cumsum(padded, axis=1)
    seg = (ends - padded).astype(jnp.float32)
    slot2 = _slot_assign(eidx, seg)
    slot0 = slot2[:, 0]
    slot1 = slot2[:, 1]
    w0 = wgt[:, 0]
    w1 = wgt[:, 1]
    blk_start = jnp.arange(ns // _T, dtype=jnp.int32) * _T
    be = jnp.minimum(
        jnp.sum((blk_start[:, None] >= ends[0, :e][None, :]).astype(
            jnp.int32), axis=1), e - 1).astype(jnp.int32)

    xs, gw = _sc_dispatch(x_flat, slot0, slot1, w0, w1, ns)
    ys = _ffn(xs, W1, b1, W2, b2, gw.reshape(ns, 1), be)
    y = _sc_combine(ys, slot0, slot1)
    return y.reshape(b, s, d)


# speedup vs baseline: 1.2161x; 1.0010x over previous
import functools

import jax
import jax.numpy as jnp
from jax import lax
from jax.experimental import pallas as pl
from jax.experimental.pallas import tpu as pltpu
from jax.experimental.pallas import tpu_sc as plsc

_TOPK = 2
_T = 256
_RT = 512


def _router_body(x_ref, wg_ref, bg_ref, eidx_ref, wgt_ref, cnt_ref):
    step = pl.program_id(0)
    x = x_ref[...]
    logits = jnp.dot(x, wg_ref[...], preferred_element_type=jnp.float32)
    logits = logits + bg_ref[...]
    m = jnp.max(logits, axis=1, keepdims=True)
    ex = jnp.exp(logits - m)
    probs = ex / jnp.sum(ex, axis=1, keepdims=True)
    lane = jax.lax.broadcasted_iota(jnp.int32, probs.shape, 1)
    big = jnp.int32(10**6)
    m0 = jnp.max(probs, axis=1, keepdims=True)
    i0 = jnp.min(jnp.where(probs == m0, lane, big), axis=1, keepdims=True)
    probs1 = jnp.where(lane == i0, -1.0, probs)
    m1 = jnp.max(probs1, axis=1, keepdims=True)
    i1 = jnp.min(jnp.where(probs1 == m1, lane, big), axis=1, keepdims=True)
    eidx_ref[...] = jnp.where(lane == 0, i0, jnp.where(lane == 1, i1, 0))
    wgt_ref[...] = jnp.where(lane == 0, m0, jnp.where(lane == 1, m1, 0.0))
    oh = ((lane == i0) | (lane == i1)).astype(jnp.int32)
    chunk_counts = jnp.sum(oh, axis=0, keepdims=True)

    @pl.when(step == 0)
    def _():
        cnt_ref[...] = chunk_counts

    @pl.when(step > 0)
    def _():
        cnt_ref[...] = cnt_ref[...] + chunk_counts


def _router(x_flat, Wg, bg):
    n, d = x_flat.shape
    e = Wg.shape[1]
    wg_pad = jnp.zeros((d, 128), jnp.float32).at[:, :e].set(Wg)
    bg_pad = jnp.full((1, 128), -1e30, jnp.float32).at[0, :e].set(bg)
    return pl.pallas_call(
        _router_body,
        grid=(n // _RT,),
        in_specs=[
            pl.BlockSpec((_RT, d), lambda i: (i, 0)),
            pl.BlockSpec((d, 128), lambda i: (0, 0)),
            pl.BlockSpec((1, 128), lambda i: (0, 0)),
        ],
        out_specs=[
            pl.BlockSpec((_RT, 128), lambda i: (i, 0)),
            pl.BlockSpec((_RT, 128), lambda i: (i, 0)),
            pl.BlockSpec((1, 128), lambda i: (0, 0)),
        ],
        out_shape=[
            jax.ShapeDtypeStruct((n, 128), jnp.int32),
            jax.ShapeDtypeStruct((n, 128), jnp.float32),
            jax.ShapeDtypeStruct((1, 128), jnp.int32),
        ],
        compiler_params=pltpu.CompilerParams(
            dimension_semantics=("arbitrary",)),
    )(x_flat, wg_pad, bg_pad)


def _slot_body(eidx_ref, seg_ref, slot_ref, run_ref):
    step = pl.program_id(0)

    @pl.when(step == 0)
    def _():
        run_ref[...] = jnp.zeros_like(run_ref)

    eidx = eidx_ref[...]
    lane = jax.lax.broadcasted_iota(jnp.int32, eidx.shape, 1)
    i0 = jnp.sum(jnp.where(lane == 0, eidx, 0), axis=1, keepdims=True)
    i1 = jnp.sum(jnp.where(lane == 1, eidx, 0), axis=1, keepdims=True)
    oh0 = (lane == i0).astype(jnp.float32)
    oh1 = (lane == i1).astype(jnp.float32)
    oh = jnp.concatenate([oh0, oh1], axis=0)
    m = 2 * _RT
    r = jax.lax.broadcasted_iota(jnp.int32, (m, m), 0)
    c = jax.lax.broadcasted_iota(jnp.int32, (m, m), 1)
    ltri = (r > c).astype(jnp.float32)
    pref = jnp.dot(ltri, oh, preferred_element_type=jnp.float32)
    base = run_ref[...] + seg_ref[...]
    slot_pe = jnp.sum(oh * (pref + base), axis=1, keepdims=True)
    s0 = slot_pe[:_RT]
    s1 = slot_pe[_RT:]
    slot_ref[...] = jnp.where(
        lane == 0, s0, jnp.where(lane == 1, s1, 0.0)).astype(jnp.int32)
    run_ref[...] = run_ref[...] + jnp.sum(oh, axis=0, keepdims=True)


def _slot_assign(eidx, seg):
    n = eidx.shape[0]
    return pl.pallas_call(
        _slot_body,
        grid=(n // _RT,),
        in_specs=[
            pl.BlockSpec((_RT, 128), lambda i: (i, 0)),
            pl.BlockSpec((1, 128), lambda i: (0, 0)),
        ],
        out_specs=pl.BlockSpec((_RT, 128), lambda i: (i, 0)),
        out_shape=jax.ShapeDtypeStruct((n, 128), jnp.int32),
        scratch_shapes=[pltpu.VMEM((1, 128), jnp.float32)],
        compiler_params=pltpu.CompilerParams(
            dimension_semantics=("arbitrary",)),
    )(eidx, seg)


def _erf(z):
    s = jnp.sign(z)
    a = jnp.abs(z)
    t = 1.0 / (1.0 + 0.3275911 * a)
    poly = t * (0.254829592 + t * (-0.284496736 + t * (1.421413741
           + t * (-1.453152027 + t * 1.061405429))))
    return s * (1.0 - poly * jnp.exp(-a * a))


def _gelu(h):
    return 0.5 * h * (1.0 + _erf(h * 0.7071067811865476))


_FTA = 4096
_DTB = 1024


def _ffn1_body(nfa, be_ref, xs_ref, w1_ref, b1_ref, h_ref, w1b_ref,
               last_ref):
    fa = pl.program_id(0)
    b = pl.program_id(1)

    @pl.when((fa == 0) & (b == 0))
    def _():
        last_ref[0] = -1

    key = be_ref[b] * nfa + fa

    @pl.when(key != last_ref[0])
    def _():
        w1b_ref[...] = w1_ref[0].astype(jnp.bfloat16)
        last_ref[0] = key

    xb = xs_ref[...].astype(jnp.bfloat16)
    hh = jnp.dot(xb, w1b_ref[...], preferred_element_type=jnp.float32)
    h_ref[...] = _gelu(hh + b1_ref[0]).astype(jnp.bfloat16)


def _ffn2_body(nd, be_ref, h_ref, w2_ref, b2_ref, gw_ref, out_ref, w2b_ref,
               last_ref):
    dd = pl.program_id(0)
    b = pl.program_id(1)

    @pl.when((dd == 0) & (b == 0))
    def _():
        last_ref[0] = -1

    key = be_ref[b] * nd + dd

    @pl.when(key != last_ref[0])
    def _():
        w2b_ref[...] = w2_ref[0].astype(jnp.bfloat16)
        last_ref[0] = key

    acc = jnp.dot(h_ref[...], w2b_ref[...],
                  preferred_element_type=jnp.float32)
    out_ref[...] = (acc + b2_ref[0]) * gw_ref[...]


def _ffn(xs, W1, b1, W2, b2, gw, be):
    ns, d = xs.shape
    e, _, f_dim = W1.shape
    nb = ns // _T
    nfa = f_dim // _FTA
    nd = d // _DTB

    spec1 = pltpu.PrefetchScalarGridSpec(
        num_scalar_prefetch=1,
        grid=(nfa, nb),
        in_specs=[
            pl.BlockSpec((_T, d), lambda fa, b, be: (b, 0)),
            pl.BlockSpec((1, d, _FTA), lambda fa, b, be: (be[b], 0, fa)),
            pl.BlockSpec((1, 1, _FTA), lambda fa, b, be: (be[b], 0, fa)),
        ],
        out_specs=pl.BlockSpec((_T, _FTA), lambda fa, b, be: (b, fa)),
        scratch_shapes=[
            pltpu.VMEM((d, _FTA), jnp.bfloat16),
            pltpu.SMEM((1,), jnp.int32),
        ],
    )
    h = pl.pallas_call(
        functools.partial(_ffn1_body, nfa),
        grid_spec=spec1,
        out_shape=jax.ShapeDtypeStruct((ns, f_dim), jnp.bfloat16),
        compiler_params=pltpu.CompilerParams(
            dimension_semantics=("arbitrary", "arbitrary")),
    )(be, xs, W1, b1.reshape(e, 1, f_dim))

    spec2 = pltpu.PrefetchScalarGridSpec(
        num_scalar_prefetch=1,
        grid=(nd, nb),
        in_specs=[
            pl.BlockSpec((_T, f_dim), lambda dd, b, be: (b, 0)),
            pl.BlockSpec((1, f_dim, _DTB), lambda dd, b, be: (be[b], 0, dd)),
            pl.BlockSpec((1, 1, _DTB), lambda dd, b, be: (be[b], 0, dd)),
            pl.BlockSpec((_T, 1), lambda dd, b, be: (b, 0)),
        ],
        out_specs=pl.BlockSpec((_T, _DTB), lambda dd, b, be: (b, dd)),
        scratch_shapes=[
            pltpu.VMEM((f_dim, _DTB), jnp.bfloat16),
            pltpu.SMEM((1,), jnp.int32),
        ],
    )
    return pl.pallas_call(
        functools.partial(_ffn2_body, nd),
        grid_spec=spec2,
        out_shape=jax.ShapeDtypeStruct((ns, d), jnp.float32),
        compiler_params=pltpu.CompilerParams(
            dimension_semantics=("arbitrary", "arbitrary")),
    )(be, h, W2, b2.reshape(e, 1, d), gw)


def _sc_dispatch(x_flat, slot0, slot1, w0, w1, ns):
    n, d = x_flat.shape
    info = plsc.get_sparse_core_info()
    nw = info.num_cores * info.num_subcores
    bw = n // nw
    chunk = 64
    steps = bw // chunk
    mesh = plsc.VectorSubcoreMesh(core_axis_name="c", subcore_axis_name="s")

    @functools.partial(
        pl.kernel,
        out_type=(
            jax.ShapeDtypeStruct((ns, d), jnp.float32),
            jax.ShapeDtypeStruct((ns,), jnp.float32),
        ),
        mesh=mesh,
        scratch_types=[
            pltpu.VMEM((chunk, d), jnp.float32),
            pltpu.VMEM((chunk,), jnp.int32),
            pltpu.VMEM((chunk,), jnp.int32),
            pltpu.VMEM((chunk,), jnp.float32),
            pltpu.VMEM((chunk,), jnp.float32),
            pltpu.SemaphoreType.DMA,
            pltpu.SemaphoreType.DMA,
            pltpu.SemaphoreType.DMA,
            pltpu.SemaphoreType.DMA,
        ],
    )
    def k(x_hbm, s0_hbm, s1_hbm, w0_hbm, w1_hbm, xs_hbm, gw_hbm,
          rows_v, i0_v, i1_v, w0_v, w1_v, sem0, sem1, semw0, semw1):
        wid = lax.axis_index("s") * info.num_cores + lax.axis_index("c")
        base = wid * bw

        @pl.loop(0, steps)
        def _(ci):
            off = base + ci * chunk
            pltpu.sync_copy(x_hbm.at[pl.ds(off, chunk)], rows_v)
            pltpu.sync_copy(s0_hbm.at[pl.ds(off, chunk)], i0_v)
            pltpu.sync_copy(s1_hbm.at[pl.ds(off, chunk)], i1_v)
            pltpu.sync_copy(w0_hbm.at[pl.ds(off, chunk)], w0_v)
            pltpu.sync_copy(w1_hbm.at[pl.ds(off, chunk)], w1_v)
            c0 = pltpu.async_copy(rows_v, xs_hbm.at[i0_v], sem0)
            c1 = pltpu.async_copy(rows_v, xs_hbm.at[i1_v], sem1)
            cw0 = pltpu.async_copy(w0_v, gw_hbm.at[i0_v], semw0)
            cw1 = pltpu.async_copy(w1_v, gw_hbm.at[i1_v], semw1)
            c0.wait()
            c1.wait()
            cw0.wait()
            cw1.wait()

    return k(x_flat, slot0, slot1, w0, w1)


def _sc_combine(ys, slot0, slot1):
    n = slot0.shape[0]
    d = ys.shape[1]
    info = plsc.get_sparse_core_info()
    nw = info.num_cores * info.num_subcores
    bw = n // nw
    chunk = 32
    steps = bw // chunk
    mesh = plsc.VectorSubcoreMesh(core_axis_name="c", subcore_axis_name="s")

    @functools.partial(
        pl.kernel,
        out_type=jax.ShapeDtypeStruct((n, d), jnp.float32),
        mesh=mesh,
        scratch_types=[
            pltpu.VMEM((chunk,), jnp.int32),
            pltpu.VMEM((chunk,), jnp.int32),
            pltpu.VMEM((chunk, d), jnp.float32),
            pltpu.VMEM((chunk, d), jnp.float32),
            pltpu.SemaphoreType.DMA,
            pltpu.SemaphoreType.DMA,
        ],
    )
    def k(ys_hbm, s0_hbm, s1_hbm, out_hbm, i0_v, i1_v, r0_v, r1_v, sem0,
          sem1):
        wid = lax.axis_index("s") * info.num_cores + lax.axis_index("c")
        base = wid * bw

        @pl.loop(0, steps)
        def _(ci):
            off = base + ci * chunk
            pltpu.sync_copy(s0_hbm.at[pl.ds(off, chunk)], i0_v)
            pltpu.sync_copy(s1_hbm.at[pl.ds(off, chunk)], i1_v)
            c0 = pltpu.async_copy(ys_hbm.at[i0_v], r0_v, sem0)
            c1 = pltpu.async_copy(ys_hbm.at[i1_v], r1_v, sem1)
            c0.wait()
            c1.wait()

            @pl.loop(0, chunk)
            def _(r):
                for j in range(d // 16):
                    sl = (r, pl.ds(j * 16, 16))
                    r0_v[sl] = r0_v[sl] + r1_v[sl]

            pltpu.sync_copy(r0_v, out_hbm.at[pl.ds(off, chunk)])

    return k(ys, slot0, slot1)


def kernel(x, Wg, bg, W1, b1, W2, b2):
    b, s, d = x.shape
    e = Wg.shape[1]
    n = b * s
    nk = n * _TOPK
    ns = nk + e * _T
    x_flat = x.reshape(n, d)

    eidx, wgt, counts = _router(x_flat, Wg, bg)

    padded = ((counts + (_T - 1)) // _T) * _T
    ends = jnp.cumsum(padded, axis=1)
    seg = (ends - padded).astype(jnp.float32)
    slot2 = _slot_assign(eidx, seg)
    slot0 = slot2[:, 0]
    slot1 = slot2[:, 1]
    w0 = wgt[:, 0]
    w1 = wgt[:, 1]
    blk_start = jnp.arange(ns // _T, dtype=jnp.int32) * _T
    be = jnp.minimum(
        jnp.sum((blk_start[:, None] >= ends[0, :e][None, :]).astype(
            jnp.int32), axis=1), e - 1).astype(jnp.int32)

    xs, gw = _sc_dispatch(x_flat, slot0, slot1, w0, w1, ns)
    ys = _ffn(xs, W1, b1, W2, b2, gw.reshape(ns, 1), be)
    y = _sc_combine(ys, slot0, slot1)
    return y.reshape(b, s, d)
